# baseline (speedup 1.0000x reference)
import functools

import jax
import jax.numpy as jnp
from jax import lax
from jax.experimental import pallas as pl
from jax.experimental.pallas import tpu as pltpu
from jax.experimental.pallas import tpu_sc as plsc

B = 16384
EMB = 32
BLK = 2048

_NC, _NS = 2, 16
_NW = _NC * _NS
_BPW = B // _NW
_CH = 128
_NCHUNK = _BPW // _CH


def _sc_gather(user_ids, movie_ids, ue_tab, me_tab, ub_tab, mb_tab):
    mesh = plsc.VectorSubcoreMesh(core_axis_name="c", subcore_axis_name="s")

    @functools.partial(
        pl.kernel, mesh=mesh,
        compiler_params=pltpu.CompilerParams(use_tc_tiling_on_sc=False),
        out_type=(
            jax.ShapeDtypeStruct((B, EMB), jnp.float32),
            jax.ShapeDtypeStruct((B, EMB), jnp.float32),
            jax.ShapeDtypeStruct((B,), jnp.float32),
            jax.ShapeDtypeStruct((B,), jnp.float32),
        ),
        scratch_types=[
            pltpu.VMEM((_NCHUNK, _CH), jnp.int32),
            pltpu.VMEM((_NCHUNK, _CH), jnp.int32),
            pltpu.VMEM((_BPW, EMB), jnp.float32),
            pltpu.VMEM((_BPW, EMB), jnp.float32),
            pltpu.VMEM((_BPW,), jnp.float32),
            pltpu.VMEM((_BPW,), jnp.float32),
            pltpu.SemaphoreType.DMA,
        ],
    )
    def k(uid_hbm, mid_hbm, ue_hbm, me_hbm, ub_hbm, mb_hbm,
          out_ue, out_me, out_ub, out_mb,
          uidx_v, midx_v, urows_v, mrows_v, ub_v, mb_v, sem):
        wid = lax.axis_index("s") * _NC + lax.axis_index("c")
        base = wid * _BPW
        for j in range(_NCHUNK):
            pltpu.sync_copy(uid_hbm.at[pl.ds(base + j * _CH, _CH)], uidx_v.at[j])
            pltpu.sync_copy(mid_hbm.at[pl.ds(base + j * _CH, _CH)], midx_v.at[j])
        copies = []
        for j in range(_NCHUNK):
            sl = pl.ds(j * _CH, _CH)
            copies.append(pltpu.async_copy(ue_hbm.at[uidx_v.at[j]], urows_v.at[sl], sem))
            copies.append(pltpu.async_copy(me_hbm.at[midx_v.at[j]], mrows_v.at[sl], sem))
            copies.append(pltpu.async_copy(ub_hbm.at[uidx_v.at[j]], ub_v.at[sl], sem))
            copies.append(pltpu.async_copy(mb_hbm.at[midx_v.at[j]], mb_v.at[sl], sem))
        for c in copies:
            c.wait()
        out_sl = pl.ds(base, _BPW)
        pltpu.sync_copy(urows_v, out_ue.at[out_sl])
        pltpu.sync_copy(mrows_v, out_me.at[out_sl])
        pltpu.sync_copy(ub_v, out_ub.at[out_sl])
        pltpu.sync_copy(mb_v, out_mb.at[out_sl])

    return k(user_ids, movie_ids, ue_tab, me_tab, ub_tab, mb_tab)


def _mlp_body(ue_ref, me_ref, ub_ref, mb_ref, gb3_ref,
              w1a_ref, w1b_ref, b1_ref, w2_ref, b2_ref, w3_ref, out_ref):
    u = ue_ref[...]
    m = me_ref[...]
    base = jnp.sum(u * m, axis=1) + ub_ref[...] + mb_ref[...] + gb3_ref[0]
    h = jnp.maximum(
        jnp.dot(u, w1a_ref[...], preferred_element_type=jnp.float32)
        + jnp.dot(m, w1b_ref[...], preferred_element_type=jnp.float32)
        + b1_ref[...], 0.0)
    h = jnp.maximum(
        jnp.dot(h, w2_ref[...], preferred_element_type=jnp.float32)
        + b2_ref[...], 0.0)
    nn = jnp.sum(h * w3_ref[...], axis=1)
    out_ref[...] = base + nn


def _mlp(ue, me, ub, mb, gb3, W1a, W1b, b1, W2, b2, w3):
    grid = (B // BLK,)
    return pl.pallas_call(
        _mlp_body,
        grid=grid,
        in_specs=[
            pl.BlockSpec((BLK, EMB), lambda i: (i, 0)),
            pl.BlockSpec((BLK, EMB), lambda i: (i, 0)),
            pl.BlockSpec((BLK,), lambda i: (i,)),
            pl.BlockSpec((BLK,), lambda i: (i,)),
            pl.BlockSpec(memory_space=pltpu.SMEM),
            pl.BlockSpec((EMB, 64), lambda i: (0, 0)),
            pl.BlockSpec((EMB, 64), lambda i: (0, 0)),
            pl.BlockSpec((1, 64), lambda i: (0, 0)),
            pl.BlockSpec((64, 32), lambda i: (0, 0)),
            pl.BlockSpec((1, 32), lambda i: (0, 0)),
            pl.BlockSpec((1, 32), lambda i: (0, 0)),
        ],
        out_specs=pl.BlockSpec((BLK,), lambda i: (i,)),
        out_shape=jax.ShapeDtypeStruct((B,), jnp.float32),
    )(ue, me, ub, mb, gb3, W1a, W1b, b1, W2, b2, w3)


def kernel(user_ids, movie_ids, user_embedding, movie_embedding, user_bias,
           movie_bias, global_bias, W1, b1, W2, b2, W3, b3):
    ue, me, ub, mb = _sc_gather(
        user_ids.astype(jnp.int32), movie_ids.astype(jnp.int32),
        user_embedding, movie_embedding,
        user_bias.reshape(-1), movie_bias.reshape(-1))
    gb3 = global_bias + b3
    return _mlp(ue, me, ub, mb, gb3,
                W1[:EMB], W1[EMB:], b1.reshape(1, 64),
                W2, b2.reshape(1, 32), W3.reshape(1, 32))
